# baseline (speedup 1.0000x reference)
import functools

import jax
import jax.numpy as jnp
from jax import lax
from jax.experimental import pallas as pl
from jax.experimental.pallas import tpu as pltpu
from jax.experimental.pallas import tpu_sc as plsc

C_IN = 128
C_OUT = 128
N_IN = 50000
N_OUT = 100000
K_VOL = 8
BN_EPS = 1e-5

NC = 2
NS = 16
CHUNK = 128
NBATCH = 33
BPER = 6
NCHUNK = NBATCH * BPER
PAIRS_PER_TILE = CHUNK * NCHUNK
PAIRS_PAD = NS * PAIRS_PER_TILE
PAIRS = K_VOL * N_IN
ACC_ROWS = 100048
ZROWS = 128
ROWS_PER_TILE_Z = ACC_ROWS // NS
ROWS_PER_TILE_F = N_OUT // NS
MM_BLK = 2000


def _deconv_body(x_ref, w_ref, z_ref):
    z_ref[...] = jnp.dot(x_ref[...], w_ref[0], preferred_element_type=jnp.float32)


def _deconv_matmul(x, w):
    nblk = N_IN // MM_BLK
    return pl.pallas_call(
        _deconv_body,
        grid=(nblk, K_VOL),
        in_specs=[
            pl.BlockSpec((MM_BLK, C_IN), lambda i, k: (i, 0)),
            pl.BlockSpec((1, C_IN, C_OUT), lambda i, k: (k, 0, 0)),
        ],
        out_specs=pl.BlockSpec((MM_BLK, C_OUT), lambda i, k: (k * nblk + i, 0)),
        out_shape=jax.ShapeDtypeStruct((K_VOL * N_IN, C_OUT), jnp.float32),
    )(x, w)


def _sc_scatter_body(zf_hbm, sd_hbm, y_hbm,
                     srcb, dstb, idxb, gbuf, zbuf, acc,
                     gsem, ssem, isem, zsem):
    c = lax.axis_index("c")
    s = lax.axis_index("s")

    def zrow(i, carry):
        zbuf[i, :] = jnp.zeros((16,), jnp.float32)
        return carry
    lax.fori_loop(0, ZROWS, zrow, 0)

    def drain_scatters(p):
        for t in range(BPER):
            pltpu.make_async_copy(zf_hbm.at[pl.ds(0, CHUNK)],
                                  gbuf.at[p, t], ssem.at[p, t]).wait()

    for j in range(K_VOL // NC):
        cgoff = c * (K_VOL // NC) + j

        zbase = s * ROWS_PER_TILE_Z
        nz = ROWS_PER_TILE_Z // ZROWS
        zds = [pltpu.async_copy(zbuf, acc.at[pl.ds(zbase + b * ZROWS, ZROWS)],
                                zsem) for b in range(nz)]
        rem = ROWS_PER_TILE_Z % ZROWS
        if rem:
            zds.append(pltpu.async_copy(
                zbuf.at[pl.ds(0, rem)],
                acc.at[pl.ds(zbase + nz * ZROWS, rem)], zsem))
        for d in zds:
            d.wait()
        plsc.subcore_barrier()

        pltpu.async_copy(sd_hbm.at[0, s, pl.ds(0, BPER)], srcb.at[0],
                         isem.at[0, 0])
        pltpu.async_copy(sd_hbm.at[1, s, pl.ds(0, BPER)], dstb.at[0],
                         isem.at[0, 1])

        def batch(bi, carry):
            p = lax.rem(bi, 2)
            np_ = 1 - p

            @pl.when(bi < NBATCH - 1)
            def _():
                off = (bi + 1) * BPER
                pltpu.async_copy(sd_hbm.at[0, s, pl.ds(off, BPER)],
                                 srcb.at[np_], isem.at[np_, 0])
                pltpu.async_copy(sd_hbm.at[1, s, pl.ds(off, BPER)],
                                 dstb.at[np_], isem.at[np_, 1])

            pltpu.make_async_copy(sd_hbm.at[0, s, pl.ds(0, BPER)],
                                  srcb.at[p], isem.at[p, 0]).wait()
            pltpu.make_async_copy(sd_hbm.at[1, s, pl.ds(0, BPER)],
                                  dstb.at[p], isem.at[p, 1]).wait()

            for t in range(BPER):
                for v in range(CHUNK // 16):
                    idxb[t, pl.ds(v * 16, 16)] = (
                        srcb[p, t, pl.ds(v * 16, 16)] + cgoff)

            @pl.when(bi > 1)
            def _():
                drain_scatters(p)

            gds = [pltpu.async_copy(zf_hbm.at[idxb.at[t]], gbuf.at[p, t],
                                    gsem.at[t]) for t in range(BPER)]
            for t in range(BPER):
                gds[t].wait()
                pltpu.async_copy(gbuf.at[p, t], acc.at[dstb.at[p, t]],
                                 ssem.at[p, t], add=True)
            return carry
        lax.fori_loop(0, NBATCH, batch, 0)
        drain_scatters(0)
        drain_scatters(1)
        plsc.subcore_barrier()

        fbase = s * ROWS_PER_TILE_F
        pltpu.sync_copy(acc.at[pl.ds(fbase, ROWS_PER_TILE_F)],
                        y_hbm.at[pl.ds(fbase, ROWS_PER_TILE_F),
                                 pl.ds(cgoff * 16, 16)])
        plsc.subcore_barrier()


def _sc_scatter(zf, sd):
    mesh = plsc.VectorSubcoreMesh(core_axis_name="c", subcore_axis_name="s",
                                  num_cores=NC, num_subcores=NS)
    f = pl.kernel(
        _sc_scatter_body,
        out_type=jax.ShapeDtypeStruct((N_OUT, C_OUT), jnp.float32),
        mesh=mesh,
        compiler_params=pltpu.CompilerParams(use_tc_tiling_on_sc=False),
        scratch_types=[
            pltpu.VMEM((2, BPER, CHUNK), jnp.int32),
            pltpu.VMEM((2, BPER, CHUNK), jnp.int32),
            pltpu.VMEM((BPER, CHUNK), jnp.int32),
            pltpu.VMEM((2, BPER, CHUNK, 16), jnp.float32),
            pltpu.VMEM((ZROWS, 16), jnp.float32),
            pltpu.VMEM_SHARED((ACC_ROWS, 16), jnp.float32),
            pltpu.SemaphoreType.DMA((BPER,)),
            pltpu.SemaphoreType.DMA((2, BPER)),
            pltpu.SemaphoreType.DMA((2, 2)),
            pltpu.SemaphoreType.DMA,
        ],
    )
    return f(zf, sd)


def _bnfuse_body(y_ref, skip_ref, gb_ref, wf_ref, out_ref, s_acc, q_acc):
    ph = pl.program_id(0)
    i = pl.program_id(1)

    @pl.when((ph == 0) & (i == 0))
    def _():
        s_acc[...] = jnp.zeros_like(s_acc)
        q_acc[...] = jnp.zeros_like(q_acc)

    @pl.when(ph == 0)
    def _():
        yb = y_ref[...]
        s_acc[...] += jnp.sum(yb, axis=0, keepdims=True)
        q_acc[...] += jnp.sum(yb * yb, axis=0, keepdims=True)

    @pl.when(ph == 1)
    def _():
        inv_n = 1.0 / N_OUT
        mean = s_acc[...] * inv_n
        var = q_acc[...] * inv_n - mean * mean
        inv = lax.rsqrt(var + BN_EPS)
        scale = gb_ref[0:1, :] * inv
        bias = gb_ref[1:2, :] - mean * scale
        h = jnp.maximum(y_ref[...] * scale + bias, 0.0)
        out_ref[...] = (
            jnp.dot(h, wf_ref[0:C_OUT, :], preferred_element_type=jnp.float32)
            + jnp.dot(skip_ref[...], wf_ref[C_OUT:, :],
                      preferred_element_type=jnp.float32))


def _bn_fuse(y, skip, gb, wf):
    nblk = N_OUT // MM_BLK
    return pl.pallas_call(
        _bnfuse_body,
        grid=(2, nblk),
        in_specs=[
            pl.BlockSpec((MM_BLK, C_OUT), lambda ph, i: (i, 0)),
            pl.BlockSpec((MM_BLK, C_OUT), lambda ph, i: (i * ph, 0)),
            pl.BlockSpec((2, C_OUT), lambda ph, i: (0, 0)),
            pl.BlockSpec((C_OUT + C_OUT, C_OUT), lambda ph, i: (0, 0)),
        ],
        out_specs=pl.BlockSpec((MM_BLK, C_OUT), lambda ph, i: (i, 0)),
        out_shape=jax.ShapeDtypeStruct((N_OUT, C_OUT), jnp.float32),
        scratch_shapes=[pltpu.VMEM((1, C_OUT), jnp.float32),
                        pltpu.VMEM((1, C_OUT), jnp.float32)],
    )(y, skip, gb, wf)


@jax.jit
def kernel(x_features, skip_features, gather_idx, scatter_idx, W_deconv,
           bn_gamma, bn_beta, W_fuse):
    gi = gather_idx.astype(jnp.int32)
    si = scatter_idx.astype(jnp.int32)

    z = _deconv_matmul(x_features, W_deconv)

    koff = jnp.arange(K_VOL, dtype=jnp.int32)[:, None] * N_IN
    src8 = ((koff + gi).reshape(-1)) * 8
    dst = si.reshape(-1)
    npad = PAIRS_PAD - PAIRS
    src8 = jnp.concatenate(
        [src8, (jnp.arange(npad, dtype=jnp.int32) % PAIRS) * 8])
    dst = jnp.concatenate(
        [dst, N_OUT + (jnp.arange(npad, dtype=jnp.int32) % (ACC_ROWS - N_OUT))])
    sd = jnp.stack([src8, dst]).reshape(2, NS, NCHUNK, CHUNK)

    zf = z.reshape(K_VOL * N_IN * 8, 16)
    y = _sc_scatter(zf, sd)

    gb = jnp.stack([bn_gamma, bn_beta])
    return _bn_fuse(y, skip_features, gb, W_fuse)


# speedup vs baseline: 1.0188x; 1.0188x over previous
import jax
import jax.numpy as jnp
from jax import lax
from jax.experimental import pallas as pl
from jax.experimental.pallas import tpu as pltpu
from jax.experimental.pallas import tpu_sc as plsc

C_IN = 128
C_OUT = 128
N_IN = 50000
N_OUT = 100000
K_VOL = 8
BN_EPS = 1e-5

NC = 2
NS = 16
CHUNK = 128
NBATCH = 20
BPER = 10
NCHUNK = NBATCH * BPER
PAIRS_PER_TILE = CHUNK * NCHUNK
PAIRS_PAD = NS * PAIRS_PER_TILE
PAIRS = K_VOL * N_IN
ACC_ROWS = 100048
ZROWS = 256
ROWS_PER_TILE_Z = ACC_ROWS // NS
ROWS_PER_TILE_F = N_OUT // NS
MM_BLK = 2000


def _deconv_body(x_ref, w_ref, z_ref):
    z_ref[...] = jnp.dot(x_ref[...], w_ref[0], preferred_element_type=jnp.float32)


def _deconv_matmul(x, w):
    nblk = N_IN // MM_BLK
    return pl.pallas_call(
        _deconv_body,
        grid=(nblk, K_VOL),
        in_specs=[
            pl.BlockSpec((MM_BLK, C_IN), lambda i, k: (i, 0)),
            pl.BlockSpec((1, C_IN, C_OUT), lambda i, k: (k, 0, 0)),
        ],
        out_specs=pl.BlockSpec((MM_BLK, C_OUT), lambda i, k: (k * nblk + i, 0)),
        out_shape=jax.ShapeDtypeStruct((K_VOL * N_IN, C_OUT), jnp.float32),
    )(x, w)


def _sc_scatter_body(zf_hbm, sd_hbm, y_hbm,
                     srcb, dstb, idxb, gbuf, zbuf, acc,
                     gsem, ssem, isem, zsem):
    c = lax.axis_index("c")
    s = lax.axis_index("s")

    def zrow(i, carry):
        zbuf[i, :] = jnp.zeros((16,), jnp.float32)
        return carry
    lax.fori_loop(0, ZROWS, zrow, 0)

    def drain_scatters():
        for t in range(BPER):
            pltpu.make_async_copy(zf_hbm.at[pl.ds(0, CHUNK)],
                                  gbuf.at[t], ssem.at[t]).wait()

    for j in range(K_VOL // NC):
        cgoff = c * (K_VOL // NC) + j

        zbase = s * ROWS_PER_TILE_Z
        nz = ROWS_PER_TILE_Z // ZROWS
        zds = [pltpu.async_copy(zbuf, acc.at[pl.ds(zbase + b * ZROWS, ZROWS)],
                                zsem) for b in range(nz)]
        rem = ROWS_PER_TILE_Z % ZROWS
        if rem:
            zds.append(pltpu.async_copy(
                zbuf.at[pl.ds(0, rem)],
                acc.at[pl.ds(zbase + nz * ZROWS, rem)], zsem))
        for d in zds:
            d.wait()
        plsc.subcore_barrier()

        pltpu.async_copy(sd_hbm.at[0, s, pl.ds(0, BPER)], srcb.at[0],
                         isem.at[0, 0])
        pltpu.async_copy(sd_hbm.at[1, s, pl.ds(0, BPER)], dstb.at[0],
                         isem.at[0, 1])

        def batch(bi, carry):
            p = lax.rem(bi, 2)
            np_ = 1 - p

            @pl.when(bi < NBATCH - 1)
            def _():
                off = (bi + 1) * BPER
                pltpu.async_copy(sd_hbm.at[0, s, pl.ds(off, BPER)],
                                 srcb.at[np_], isem.at[np_, 0])
                pltpu.async_copy(sd_hbm.at[1, s, pl.ds(off, BPER)],
                                 dstb.at[np_], isem.at[np_, 1])

            pltpu.make_async_copy(sd_hbm.at[0, s, pl.ds(0, BPER)],
                                  srcb.at[p], isem.at[p, 0]).wait()
            pltpu.make_async_copy(sd_hbm.at[1, s, pl.ds(0, BPER)],
                                  dstb.at[p], isem.at[p, 1]).wait()

            @pl.when(bi > 0)
            def _():
                drain_scatters()

            for t in range(BPER):
                for v in range(CHUNK // 16):
                    idxb[t, pl.ds(v * 16, 16)] = (
                        srcb[p, t, pl.ds(v * 16, 16)] + cgoff)
            gds = [pltpu.async_copy(zf_hbm.at[idxb.at[t]], gbuf.at[t],
                                    gsem.at[t]) for t in range(BPER)]
            for t in range(BPER):
                gds[t].wait()
                pltpu.async_copy(gbuf.at[t], acc.at[dstb.at[p, t]],
                                 ssem.at[t], add=True)
            return carry
        lax.fori_loop(0, NBATCH, batch, 0)
        drain_scatters()
        plsc.subcore_barrier()

        fbase = s * ROWS_PER_TILE_F
        pltpu.sync_copy(acc.at[pl.ds(fbase, ROWS_PER_TILE_F)],
                        y_hbm.at[pl.ds(fbase, ROWS_PER_TILE_F),
                                 pl.ds(cgoff * 16, 16)])
        plsc.subcore_barrier()


def _sc_scatter(zf, sd):
    mesh = plsc.VectorSubcoreMesh(core_axis_name="c", subcore_axis_name="s",
                                  num_cores=NC, num_subcores=NS)
    f = pl.kernel(
        _sc_scatter_body,
        out_type=jax.ShapeDtypeStruct((N_OUT, C_OUT), jnp.float32),
        mesh=mesh,
        compiler_params=pltpu.CompilerParams(use_tc_tiling_on_sc=False),
        scratch_types=[
            pltpu.VMEM((2, BPER, CHUNK), jnp.int32),
            pltpu.VMEM((2, BPER, CHUNK), jnp.int32),
            pltpu.VMEM((BPER, CHUNK), jnp.int32),
            pltpu.VMEM((BPER, CHUNK, 16), jnp.float32),
            pltpu.VMEM((ZROWS, 16), jnp.float32),
            pltpu.VMEM_SHARED((ACC_ROWS, 16), jnp.float32),
            pltpu.SemaphoreType.DMA((BPER,)),
            pltpu.SemaphoreType.DMA((BPER,)),
            pltpu.SemaphoreType.DMA((2, 2)),
            pltpu.SemaphoreType.DMA,
        ],
    )
    return f(zf, sd)


def _bnfuse_body(y_ref, skip_ref, gb_ref, wf_ref, out_ref, s_acc, q_acc):
    ph = pl.program_id(0)
    i = pl.program_id(1)

    @pl.when((ph == 0) & (i == 0))
    def _():
        s_acc[...] = jnp.zeros_like(s_acc)
        q_acc[...] = jnp.zeros_like(q_acc)

    @pl.when(ph == 0)
    def _():
        yb = y_ref[...]
        s_acc[...] += jnp.sum(yb, axis=0, keepdims=True)
        q_acc[...] += jnp.sum(yb * yb, axis=0, keepdims=True)

    @pl.when(ph == 1)
    def _():
        inv_n = 1.0 / N_OUT
        mean = s_acc[...] * inv_n
        var = q_acc[...] * inv_n - mean * mean
        inv = lax.rsqrt(var + BN_EPS)
        scale = gb_ref[0:1, :] * inv
        bias = gb_ref[1:2, :] - mean * scale
        h = jnp.maximum(y_ref[...] * scale + bias, 0.0)
        out_ref[...] = (
            jnp.dot(h, wf_ref[0:C_OUT, :], preferred_element_type=jnp.float32)
            + jnp.dot(skip_ref[...], wf_ref[C_OUT:, :],
                      preferred_element_type=jnp.float32))


def _bn_fuse(y, skip, gb, wf):
    nblk = N_OUT // MM_BLK
    return pl.pallas_call(
        _bnfuse_body,
        grid=(2, nblk),
        in_specs=[
            pl.BlockSpec((MM_BLK, C_OUT), lambda ph, i: (i, 0)),
            pl.BlockSpec((MM_BLK, C_OUT), lambda ph, i: (i * ph, 0)),
            pl.BlockSpec((2, C_OUT), lambda ph, i: (0, 0)),
            pl.BlockSpec((C_OUT + C_OUT, C_OUT), lambda ph, i: (0, 0)),
        ],
        out_specs=pl.BlockSpec((MM_BLK, C_OUT), lambda ph, i: (i, 0)),
        out_shape=jax.ShapeDtypeStruct((N_OUT, C_OUT), jnp.float32),
        scratch_shapes=[pltpu.VMEM((1, C_OUT), jnp.float32),
                        pltpu.VMEM((1, C_OUT), jnp.float32)],
    )(y, skip, gb, wf)


@jax.jit
def kernel(x_features, skip_features, gather_idx, scatter_idx, W_deconv,
           bn_gamma, bn_beta, W_fuse):
    gi = gather_idx.astype(jnp.int32)
    si = scatter_idx.astype(jnp.int32)

    z = _deconv_matmul(x_features, W_deconv)

    koff = jnp.arange(K_VOL, dtype=jnp.int32)[:, None] * N_IN
    src8 = ((koff + gi).reshape(-1)) * 8
    dst = si.reshape(-1)
    npad = PAIRS_PAD - PAIRS
    src8 = jnp.concatenate(
        [src8, (jnp.arange(npad, dtype=jnp.int32) % PAIRS) * 8])
    dst = jnp.concatenate(
        [dst, N_OUT + (jnp.arange(npad, dtype=jnp.int32) % (ACC_ROWS - N_OUT))])
    sd = jnp.stack([src8, dst]).reshape(2, NS, NCHUNK, CHUNK)

    zf = z.reshape(K_VOL * N_IN * 8, 16)
    y = _sc_scatter(zf, sd)

    gb = jnp.stack([bn_gamma, bn_beta])
    return _bn_fuse(y, skip_features, gb, W_fuse)
